# baseline (device time: 461831 ns/iter reference)
import jax
import jax.numpy as jnp
from jax import lax
from jax.experimental import pallas as pl
from jax.experimental.pallas import tpu as pltpu

T = 4096
D = 2048
F = 4096
E_LOCAL = 4
E_TOTAL = 8
CAP = 544
HALF = E_LOCAL * CAP
FTILE = 512
STILE = CAP
TTILE = 512


def _xchg_body(src_ref, dst_ref, send_sem, recv_sem):
    peer = (lax.axis_index("x"), 1 - lax.axis_index("y"))
    barrier = pltpu.get_barrier_semaphore()
    pl.semaphore_signal(
        barrier, inc=1, device_id=peer, device_id_type=pl.DeviceIdType.MESH
    )
    pl.semaphore_wait(barrier, 1)
    rdma = pltpu.make_async_remote_copy(
        src_ref=src_ref,
        dst_ref=dst_ref,
        send_sem=send_sem,
        recv_sem=recv_sem,
        device_id=peer,
        device_id_type=pl.DeviceIdType.MESH,
    )
    rdma.start()
    rdma.wait()


def _exchange(buf, collective_id):
    return pl.pallas_call(
        _xchg_body,
        out_shape=jax.ShapeDtypeStruct(buf.shape, buf.dtype),
        in_specs=[pl.BlockSpec(memory_space=pltpu.VMEM)],
        out_specs=pl.BlockSpec(memory_space=pltpu.VMEM),
        scratch_shapes=[pltpu.SemaphoreType.DMA, pltpu.SemaphoreType.DMA],
        compiler_params=pltpu.CompilerParams(collective_id=collective_id),
    )(buf)


def _dispatch_body(
    ts_ref, x_ref, rows_ref, theirs_buf, send_sems, recv_sems
):
    peer = (lax.axis_index("x"), 1 - lax.axis_index("y"))
    barrier = pltpu.get_barrier_semaphore()
    pl.semaphore_signal(
        barrier, inc=1, device_id=peer, device_id_type=pl.DeviceIdType.MESH
    )
    pl.semaphore_wait(barrier, 1)

    nt = HALF // STILE

    def onehot_rows(base, t):
        rows = (base + t * STILE) + lax.broadcasted_iota(
            jnp.int32, (STILE, 1), 0
        )
        return (rows == ts_ref[...]).astype(jnp.bfloat16)

    rdmas = []
    for t in range(nt):
        theirs_buf[pl.ds(t * STILE, STILE), :] = jnp.dot(
            onehot_rows(HALF, t), x_ref[...],
            preferred_element_type=jnp.float32,
        ).astype(jnp.bfloat16)
        rdma = pltpu.make_async_remote_copy(
            src_ref=theirs_buf.at[pl.ds(t * STILE, STILE), :],
            dst_ref=rows_ref.at[pl.ds(t * 2 * CAP + CAP, CAP), :],
            send_sem=send_sems.at[t],
            recv_sem=recv_sems.at[t],
            device_id=peer,
            device_id_type=pl.DeviceIdType.MESH,
        )
        rdma.start()
        rdmas.append(rdma)

    for t in range(nt):
        rows_ref[pl.ds(t * 2 * CAP, STILE), :] = jnp.dot(
            onehot_rows(0, t), x_ref[...],
            preferred_element_type=jnp.float32,
        ).astype(jnp.bfloat16)

    for rdma in rdmas:
        rdma.wait()


def _dispatch(x_bf, ts_row):
    return pl.pallas_call(
        _dispatch_body,
        out_shape=jax.ShapeDtypeStruct((2 * HALF, D), jnp.bfloat16),
        in_specs=[
            pl.BlockSpec(memory_space=pltpu.VMEM),
            pl.BlockSpec(memory_space=pltpu.VMEM),
        ],
        out_specs=pl.BlockSpec(memory_space=pltpu.VMEM),
        scratch_shapes=[
            pltpu.VMEM((HALF, D), jnp.bfloat16),
            pltpu.SemaphoreType.DMA((HALF // STILE,)),
            pltpu.SemaphoreType.DMA((HALF // STILE,)),
        ],
        compiler_params=pltpu.CompilerParams(
            collective_id=0, vmem_limit_bytes=100 * 1024 * 1024
        ),
    )(ts_row, x_bf)


def _ret_rdma(retbuf, returned_ref, send_sems, recv_sems, peer, e):
    return pltpu.make_async_remote_copy(
        src_ref=retbuf.at[pl.ds(e * CAP, CAP), :],
        dst_ref=returned_ref.at[pl.ds(e * CAP, CAP), :],
        send_sem=send_sems.at[e],
        recv_sem=recv_sems.at[e],
        device_id=peer,
        device_id_type=pl.DeviceIdType.MESH,
    )


def _moe_body(
    rows_ref,
    w1_ref,
    w2_ref,
    outm_ref,
    returned_ref,
    acc,
    retbuf,
    send_sems,
    recv_sems,
):
    e = pl.program_id(0)
    ft = pl.program_id(1)
    peer = (lax.axis_index("x"), 1 - lax.axis_index("y"))

    @pl.when((e == 0) & (ft == 0))
    def _():
        barrier = pltpu.get_barrier_semaphore()
        pl.semaphore_signal(
            barrier,
            inc=1,
            device_id=peer,
            device_id_type=pl.DeviceIdType.MESH,
        )
        pl.semaphore_wait(barrier, 1)

    w1 = w1_ref[0].astype(jnp.bfloat16)
    w2 = w2_ref[0].astype(jnp.bfloat16)

    h = jnp.dot(rows_ref[0], w1, preferred_element_type=jnp.float32)
    p = jnp.dot(
        jnp.maximum(h, 0.0).astype(jnp.bfloat16),
        w2,
        preferred_element_type=jnp.float32,
    )

    @pl.when(ft == 0)
    def _():
        acc[...] = p

    @pl.when(ft != 0)
    def _():
        acc[...] += p

    @pl.when(ft == F // FTILE - 1)
    def _():
        outm_ref[0] = acc[:CAP, :].astype(jnp.bfloat16)
        retbuf[pl.ds(e * CAP, CAP), :] = acc[CAP:, :].astype(jnp.bfloat16)
        _ret_rdma(retbuf, returned_ref, send_sems, recv_sems, peer, e).start()

    @pl.when((e == E_LOCAL - 1) & (ft == F // FTILE - 1))
    def _():
        for t in range(E_LOCAL):
            _ret_rdma(
                retbuf, returned_ref, send_sems, recv_sems, peer, t
            ).wait()


def _moe(rows_all, w1, w2):
    return pl.pallas_call(
        _moe_body,
        grid=(E_LOCAL, F // FTILE),
        in_specs=[
            pl.BlockSpec((1, 2 * CAP, D), lambda e, f: (e, 0, 0)),
            pl.BlockSpec((1, D, FTILE), lambda e, f: (e, 0, f)),
            pl.BlockSpec((1, FTILE, D), lambda e, f: (e, f, 0)),
        ],
        out_specs=[
            pl.BlockSpec((1, CAP, D), lambda e, f: (e, 0, 0)),
            pl.BlockSpec(memory_space=pltpu.VMEM),
        ],
        out_shape=[
            jax.ShapeDtypeStruct((E_LOCAL, CAP, D), jnp.bfloat16),
            jax.ShapeDtypeStruct((HALF, D), jnp.bfloat16),
        ],
        scratch_shapes=[
            pltpu.VMEM((2 * CAP, D), jnp.float32),
            pltpu.VMEM((HALF, D), jnp.bfloat16),
            pltpu.SemaphoreType.DMA((E_LOCAL,)),
            pltpu.SemaphoreType.DMA((E_LOCAL,)),
        ],
        compiler_params=pltpu.CompilerParams(
            collective_id=1, vmem_limit_bytes=100 * 1024 * 1024
        ),
    )(rows_all, w1, w2)


def _combine_body(ts_ref, m_ref, r_ref, out_ref):
    cols = lax.broadcasted_iota(jnp.int32, (1, HALF), 1)
    ts = ts_ref[...]
    pm = (cols == ts).astype(jnp.bfloat16)
    pr = (cols == ts - HALF).astype(jnp.bfloat16)
    out_ref[...] = jnp.dot(
        pm, m_ref[...], preferred_element_type=jnp.float32
    ) + jnp.dot(pr, r_ref[...], preferred_element_type=jnp.float32)


def _combine(out_mine, returned, ts_col):
    return pl.pallas_call(
        _combine_body,
        grid=(T // TTILE,),
        in_specs=[
            pl.BlockSpec((TTILE, 1), lambda j: (j, 0)),
            pl.BlockSpec((HALF, D), lambda j: (0, 0)),
            pl.BlockSpec((HALF, D), lambda j: (0, 0)),
        ],
        out_specs=pl.BlockSpec((TTILE, D), lambda j: (j, 0)),
        out_shape=jax.ShapeDtypeStruct((T, D), jnp.float32),
        compiler_params=pltpu.CompilerParams(
            vmem_limit_bytes=100 * 1024 * 1024
        ),
    )(ts_col, out_mine, returned)


def kernel(x, assign, W1, W2):
    my_y = lax.axis_index("y")

    onehot = (
        assign[:, None] == jnp.arange(E_TOTAL, dtype=jnp.int32)[None, :]
    ).astype(jnp.int32)
    ranks = jnp.sum(onehot * jnp.cumsum(onehot, axis=0), axis=1) - 1
    section = jnp.where(assign // E_LOCAL == my_y, 0, HALF)
    token_slot = section + (assign % E_LOCAL) * CAP + ranks

    rows_all = _dispatch(
        x.astype(jnp.bfloat16), token_slot.reshape(1, T)
    )

    out_mine, returned = _moe(
        rows_all.reshape(E_LOCAL, 2 * CAP, D), W1, W2
    )

    return _combine(
        out_mine.reshape(HALF, D), returned, token_slot.reshape(T, 1)
    )


# device time: 461100 ns/iter; 1.0016x vs baseline; 1.0016x over previous
import jax
import jax.numpy as jnp
from jax import lax
from jax.experimental import pallas as pl
from jax.experimental.pallas import tpu as pltpu

T = 4096
D = 2048
F = 4096
E_LOCAL = 4
E_TOTAL = 8
CAP = 544
HALF = E_LOCAL * CAP
FTILE = 512
STILE = CAP
TTILE = 1024


def _xchg_body(src_ref, dst_ref, send_sem, recv_sem):
    peer = (lax.axis_index("x"), 1 - lax.axis_index("y"))
    barrier = pltpu.get_barrier_semaphore()
    pl.semaphore_signal(
        barrier, inc=1, device_id=peer, device_id_type=pl.DeviceIdType.MESH
    )
    pl.semaphore_wait(barrier, 1)
    rdma = pltpu.make_async_remote_copy(
        src_ref=src_ref,
        dst_ref=dst_ref,
        send_sem=send_sem,
        recv_sem=recv_sem,
        device_id=peer,
        device_id_type=pl.DeviceIdType.MESH,
    )
    rdma.start()
    rdma.wait()


def _exchange(buf, collective_id):
    return pl.pallas_call(
        _xchg_body,
        out_shape=jax.ShapeDtypeStruct(buf.shape, buf.dtype),
        in_specs=[pl.BlockSpec(memory_space=pltpu.VMEM)],
        out_specs=pl.BlockSpec(memory_space=pltpu.VMEM),
        scratch_shapes=[pltpu.SemaphoreType.DMA, pltpu.SemaphoreType.DMA],
        compiler_params=pltpu.CompilerParams(collective_id=collective_id),
    )(buf)


def _dispatch_body(
    ts_ref, x_ref, rows_ref, theirs_buf, send_sems, recv_sems
):
    peer = (lax.axis_index("x"), 1 - lax.axis_index("y"))
    barrier = pltpu.get_barrier_semaphore()
    pl.semaphore_signal(
        barrier, inc=1, device_id=peer, device_id_type=pl.DeviceIdType.MESH
    )
    pl.semaphore_wait(barrier, 1)

    nt = HALF // STILE

    def onehot_rows(base, t):
        rows = (base + t * STILE) + lax.broadcasted_iota(
            jnp.int32, (STILE, 1), 0
        )
        return (rows == ts_ref[...]).astype(jnp.bfloat16)

    rdmas = []
    for t in range(nt):
        theirs_buf[pl.ds(t * STILE, STILE), :] = jnp.dot(
            onehot_rows(HALF, t), x_ref[...],
            preferred_element_type=jnp.float32,
        ).astype(jnp.bfloat16)
        rdma = pltpu.make_async_remote_copy(
            src_ref=theirs_buf.at[pl.ds(t * STILE, STILE), :],
            dst_ref=rows_ref.at[pl.ds(t * 2 * CAP + CAP, CAP), :],
            send_sem=send_sems.at[t],
            recv_sem=recv_sems.at[t],
            device_id=peer,
            device_id_type=pl.DeviceIdType.MESH,
        )
        rdma.start()
        rdmas.append(rdma)

    def onehot_mine(t):
        rows = t * 2 * STILE + lax.broadcasted_iota(
            jnp.int32, (2 * STILE, 1), 0
        )
        return (rows == ts_ref[...]).astype(jnp.bfloat16)

    for t in range(nt // 2):
        pair = jnp.dot(
            onehot_mine(t), x_ref[...],
            preferred_element_type=jnp.float32,
        ).astype(jnp.bfloat16)
        rows_ref[pl.ds((2 * t) * 2 * CAP, CAP), :] = pair[:CAP, :]
        rows_ref[pl.ds((2 * t + 1) * 2 * CAP, CAP), :] = pair[CAP:, :]

    for rdma in rdmas:
        rdma.wait()


def _dispatch(x_bf, ts_row):
    return pl.pallas_call(
        _dispatch_body,
        out_shape=jax.ShapeDtypeStruct((2 * HALF, D), jnp.bfloat16),
        in_specs=[
            pl.BlockSpec(memory_space=pltpu.VMEM),
            pl.BlockSpec(memory_space=pltpu.VMEM),
        ],
        out_specs=pl.BlockSpec(memory_space=pltpu.VMEM),
        scratch_shapes=[
            pltpu.VMEM((HALF, D), jnp.bfloat16),
            pltpu.SemaphoreType.DMA((HALF // STILE,)),
            pltpu.SemaphoreType.DMA((HALF // STILE,)),
        ],
        compiler_params=pltpu.CompilerParams(
            collective_id=0, vmem_limit_bytes=100 * 1024 * 1024
        ),
    )(ts_row, x_bf)


def _ret_rdma(retbuf, returned_ref, send_sems, recv_sems, peer, e):
    return pltpu.make_async_remote_copy(
        src_ref=retbuf.at[pl.ds(e * CAP, CAP), :],
        dst_ref=returned_ref.at[pl.ds(e * CAP, CAP), :],
        send_sem=send_sems.at[e],
        recv_sem=recv_sems.at[e],
        device_id=peer,
        device_id_type=pl.DeviceIdType.MESH,
    )


def _moe_body(
    rows_ref,
    w1_ref,
    w2_ref,
    outm_ref,
    returned_ref,
    acc,
    retbuf,
    send_sems,
    recv_sems,
):
    e = pl.program_id(0)
    ft = pl.program_id(1)
    peer = (lax.axis_index("x"), 1 - lax.axis_index("y"))

    @pl.when((e == 0) & (ft == 0))
    def _():
        barrier = pltpu.get_barrier_semaphore()
        pl.semaphore_signal(
            barrier,
            inc=1,
            device_id=peer,
            device_id_type=pl.DeviceIdType.MESH,
        )
        pl.semaphore_wait(barrier, 1)

    w1 = w1_ref[0].astype(jnp.bfloat16)
    w2 = w2_ref[0].astype(jnp.bfloat16)

    h = jnp.dot(rows_ref[0], w1, preferred_element_type=jnp.float32)
    p = jnp.dot(
        jnp.maximum(h, 0.0).astype(jnp.bfloat16),
        w2,
        preferred_element_type=jnp.float32,
    )

    @pl.when(ft == 0)
    def _():
        acc[...] = p

    @pl.when(ft != 0)
    def _():
        acc[...] += p

    @pl.when(ft == F // FTILE - 1)
    def _():
        outm_ref[0] = acc[:CAP, :].astype(jnp.bfloat16)
        retbuf[pl.ds(e * CAP, CAP), :] = acc[CAP:, :].astype(jnp.bfloat16)
        _ret_rdma(retbuf, returned_ref, send_sems, recv_sems, peer, e).start()

    @pl.when((e == E_LOCAL - 1) & (ft == F // FTILE - 1))
    def _():
        for t in range(E_LOCAL):
            _ret_rdma(
                retbuf, returned_ref, send_sems, recv_sems, peer, t
            ).wait()


def _moe(rows_all, w1, w2):
    return pl.pallas_call(
        _moe_body,
        grid=(E_LOCAL, F // FTILE),
        in_specs=[
            pl.BlockSpec((1, 2 * CAP, D), lambda e, f: (e, 0, 0)),
            pl.BlockSpec((1, D, FTILE), lambda e, f: (e, 0, f)),
            pl.BlockSpec((1, FTILE, D), lambda e, f: (e, f, 0)),
        ],
        out_specs=[
            pl.BlockSpec((1, CAP, D), lambda e, f: (e, 0, 0)),
            pl.BlockSpec(memory_space=pltpu.VMEM),
        ],
        out_shape=[
            jax.ShapeDtypeStruct((E_LOCAL, CAP, D), jnp.bfloat16),
            jax.ShapeDtypeStruct((HALF, D), jnp.bfloat16),
        ],
        scratch_shapes=[
            pltpu.VMEM((2 * CAP, D), jnp.float32),
            pltpu.VMEM((HALF, D), jnp.bfloat16),
            pltpu.SemaphoreType.DMA((E_LOCAL,)),
            pltpu.SemaphoreType.DMA((E_LOCAL,)),
        ],
        compiler_params=pltpu.CompilerParams(
            collective_id=1, vmem_limit_bytes=100 * 1024 * 1024
        ),
    )(rows_all, w1, w2)


def _combine_body(ts_ref, m_ref, r_ref, out_ref):
    cols = lax.broadcasted_iota(jnp.int32, (1, HALF), 1)
    ts = ts_ref[...]
    pm = (cols == ts).astype(jnp.bfloat16)
    pr = (cols == ts - HALF).astype(jnp.bfloat16)
    out_ref[...] = jnp.dot(
        pm, m_ref[...], preferred_element_type=jnp.float32
    ) + jnp.dot(pr, r_ref[...], preferred_element_type=jnp.float32)


def _combine(out_mine, returned, ts_col):
    return pl.pallas_call(
        _combine_body,
        grid=(T // TTILE,),
        in_specs=[
            pl.BlockSpec((TTILE, 1), lambda j: (j, 0)),
            pl.BlockSpec((HALF, D), lambda j: (0, 0)),
            pl.BlockSpec((HALF, D), lambda j: (0, 0)),
        ],
        out_specs=pl.BlockSpec((TTILE, D), lambda j: (j, 0)),
        out_shape=jax.ShapeDtypeStruct((T, D), jnp.float32),
        compiler_params=pltpu.CompilerParams(
            vmem_limit_bytes=100 * 1024 * 1024
        ),
    )(ts_col, out_mine, returned)


def kernel(x, assign, W1, W2):
    my_y = lax.axis_index("y")

    onehot = (
        assign[:, None] == jnp.arange(E_TOTAL, dtype=jnp.int32)[None, :]
    ).astype(jnp.int32)
    ranks = jnp.sum(onehot * jnp.cumsum(onehot, axis=0), axis=1) - 1
    section = jnp.where(assign // E_LOCAL == my_y, 0, HALF)
    token_slot = section + (assign % E_LOCAL) * CAP + ranks

    rows_all = _dispatch(
        x.astype(jnp.bfloat16), token_slot.reshape(1, T)
    )

    out_mine, returned = _moe(
        rows_all.reshape(E_LOCAL, 2 * CAP, D), W1, W2
    )

    return _combine(
        out_mine.reshape(HALF, D), returned, token_slot.reshape(T, 1)
    )


# device time: 416509 ns/iter; 1.1088x vs baseline; 1.1071x over previous
import jax
import jax.numpy as jnp
from jax import lax
from jax.experimental import pallas as pl
from jax.experimental.pallas import tpu as pltpu

T = 4096
D = 2048
F = 4096
E_LOCAL = 4
E_TOTAL = 8
CAP = 544
HALF = E_LOCAL * CAP
FTILE = 512
STILE = CAP
TTILE = 1024


def _xchg_body(src_ref, dst_ref, send_sem, recv_sem):
    peer = (lax.axis_index("x"), 1 - lax.axis_index("y"))
    barrier = pltpu.get_barrier_semaphore()
    pl.semaphore_signal(
        barrier, inc=1, device_id=peer, device_id_type=pl.DeviceIdType.MESH
    )
    pl.semaphore_wait(barrier, 1)
    rdma = pltpu.make_async_remote_copy(
        src_ref=src_ref,
        dst_ref=dst_ref,
        send_sem=send_sem,
        recv_sem=recv_sem,
        device_id=peer,
        device_id_type=pl.DeviceIdType.MESH,
    )
    rdma.start()
    rdma.wait()


def _exchange(buf, collective_id):
    return pl.pallas_call(
        _xchg_body,
        out_shape=jax.ShapeDtypeStruct(buf.shape, buf.dtype),
        in_specs=[pl.BlockSpec(memory_space=pltpu.VMEM)],
        out_specs=pl.BlockSpec(memory_space=pltpu.VMEM),
        scratch_shapes=[pltpu.SemaphoreType.DMA, pltpu.SemaphoreType.DMA],
        compiler_params=pltpu.CompilerParams(collective_id=collective_id),
    )(buf)


def _dispatch_body(
    ts_ref, x_ref, rows_ref, theirs_buf, send_sems, recv_sems
):
    peer = (lax.axis_index("x"), 1 - lax.axis_index("y"))
    barrier = pltpu.get_barrier_semaphore()
    pl.semaphore_signal(
        barrier, inc=1, device_id=peer, device_id_type=pl.DeviceIdType.MESH
    )
    pl.semaphore_wait(barrier, 1)

    nt = HALF // STILE

    def onehot_rows(base, t):
        rows = (base + t * STILE) + lax.broadcasted_iota(
            jnp.int32, (STILE, 1), 0
        )
        return (rows == ts_ref[...]).astype(jnp.bfloat16)

    rdmas = []
    for t in range(nt):
        theirs_buf[pl.ds(t * STILE, STILE), :] = jnp.dot(
            onehot_rows(HALF, t), x_ref[...],
            preferred_element_type=jnp.float32,
        ).astype(jnp.bfloat16)
        rdma = pltpu.make_async_remote_copy(
            src_ref=theirs_buf.at[pl.ds(t * STILE, STILE), :],
            dst_ref=rows_ref.at[pl.ds(t * 2 * CAP + CAP, CAP), :],
            send_sem=send_sems.at[t],
            recv_sem=recv_sems.at[t],
            device_id=peer,
            device_id_type=pl.DeviceIdType.MESH,
        )
        rdma.start()
        rdmas.append(rdma)

    def onehot_mine(t):
        rows = t * 2 * STILE + lax.broadcasted_iota(
            jnp.int32, (2 * STILE, 1), 0
        )
        return (rows == ts_ref[...]).astype(jnp.bfloat16)

    for t in range(nt // 2):
        pair = jnp.dot(
            onehot_mine(t), x_ref[...],
            preferred_element_type=jnp.float32,
        ).astype(jnp.bfloat16)
        rows_ref[pl.ds((2 * t) * 2 * CAP, CAP), :] = pair[:CAP, :]
        rows_ref[pl.ds((2 * t + 1) * 2 * CAP, CAP), :] = pair[CAP:, :]

    for rdma in rdmas:
        rdma.wait()


def _dispatch(x_bf, ts_row):
    return pl.pallas_call(
        _dispatch_body,
        out_shape=jax.ShapeDtypeStruct((2 * HALF, D), jnp.bfloat16),
        in_specs=[
            pl.BlockSpec(memory_space=pltpu.VMEM),
            pl.BlockSpec(memory_space=pltpu.VMEM),
        ],
        out_specs=pl.BlockSpec(memory_space=pltpu.VMEM),
        scratch_shapes=[
            pltpu.VMEM((HALF, D), jnp.bfloat16),
            pltpu.SemaphoreType.DMA((HALF // STILE,)),
            pltpu.SemaphoreType.DMA((HALF // STILE,)),
        ],
        compiler_params=pltpu.CompilerParams(
            collective_id=0, vmem_limit_bytes=100 * 1024 * 1024
        ),
    )(ts_row, x_bf)


def _flow(src_ref, src_row, dst_ref, dst_row, ssem, rsem, target, e):
    return pltpu.make_async_remote_copy(
        src_ref=src_ref.at[pl.ds(src_row, CAP), :],
        dst_ref=dst_ref.at[pl.ds(dst_row, CAP), :],
        send_sem=ssem.at[e],
        recv_sem=rsem.at[e],
        device_id=target,
        device_id_type=pl.DeviceIdType.MESH,
    )


def _moe_body(
    rows_ref,
    w1_ref,
    w2_ref,
    outm_ref,
    returned_ref,
    acc,
    retbuf,
    s1,
    r1,
    s2,
    r2,
    s3,
    r3,
):
    e = pl.program_id(0)
    ft = pl.program_id(1)
    my_x = lax.axis_index("x")
    my_y = lax.axis_index("y")
    xnbr = (1 - my_x, my_y)
    ypeer = (my_x, 1 - my_y)
    last = F // FTILE - 1

    @pl.when((e == 0) & (ft == 0))
    def _():
        barrier = pltpu.get_barrier_semaphore()
        for tgt in (xnbr, ypeer):
            pl.semaphore_signal(
                barrier,
                inc=1,
                device_id=tgt,
                device_id_type=pl.DeviceIdType.MESH,
            )
        pl.semaphore_wait(barrier, 2)

    w1 = w1_ref[0].astype(jnp.bfloat16)
    w2 = w2_ref[0].astype(jnp.bfloat16)

    h = jnp.dot(rows_ref[0], w1, preferred_element_type=jnp.float32)
    p = jnp.dot(
        jnp.maximum(h, 0.0).astype(jnp.bfloat16),
        w2,
        preferred_element_type=jnp.float32,
    )

    @pl.when(ft == 0)
    def _():
        acc[...] = p

    @pl.when(ft != 0)
    def _():
        acc[...] += p

    @pl.when(ft == last)
    def _():
        res = acc[...].astype(jnp.bfloat16)

        @pl.when(my_x == 0)
        def _():
            outm_ref[pl.ds(e * CAP, CAP), :] = res
            _flow(outm_ref, e * CAP, outm_ref, e * CAP, s1, r1, xnbr, e).start()

        @pl.when(my_x == 1)
        def _():
            retbuf[pl.ds(e * CAP, CAP), :] = res
            _flow(
                retbuf, e * CAP, returned_ref, e * CAP, s2, r2, ypeer, e
            ).start()

            @pl.when(e > 0)
            def _():
                prev = (e - 1) * CAP
                _flow(
                    retbuf, prev, returned_ref, prev, s2, r2, ypeer, e - 1
                ).wait_recv()
                _flow(
                    returned_ref, prev, returned_ref, prev, s3, r3, xnbr,
                    e - 1,
                ).start()

    @pl.when((e == E_LOCAL - 1) & (ft == last))
    def _():
        @pl.when(my_x == 0)
        def _():
            for t in range(E_LOCAL):
                row = t * CAP
                _flow(
                    outm_ref, row, outm_ref, row, s1, r1, xnbr, t
                ).wait_send()
                _flow(
                    returned_ref, row, returned_ref, row, s3, r3, xnbr, t
                ).wait_recv()

        @pl.when(my_x == 1)
        def _():
            row = (E_LOCAL - 1) * CAP
            _flow(
                retbuf, row, returned_ref, row, s2, r2, ypeer, E_LOCAL - 1
            ).wait_recv()
            _flow(
                returned_ref, row, returned_ref, row, s3, r3, xnbr,
                E_LOCAL - 1,
            ).start()
            for t in range(E_LOCAL):
                row = t * CAP
                _flow(
                    retbuf, row, returned_ref, row, s2, r2, ypeer, t
                ).wait_send()
                _flow(
                    returned_ref, row, returned_ref, row, s3, r3, xnbr, t
                ).wait_send()
                _flow(
                    outm_ref, row, outm_ref, row, s1, r1, xnbr, t
                ).wait_recv()


def _moe(rows_half, w1, w2):
    return pl.pallas_call(
        _moe_body,
        grid=(E_LOCAL, F // FTILE),
        in_specs=[
            pl.BlockSpec((1, CAP, D), lambda e, f: (e, 0, 0)),
            pl.BlockSpec((1, D, FTILE), lambda e, f: (e, 0, f)),
            pl.BlockSpec((1, FTILE, D), lambda e, f: (e, f, 0)),
        ],
        out_specs=[
            pl.BlockSpec(memory_space=pltpu.VMEM),
            pl.BlockSpec(memory_space=pltpu.VMEM),
        ],
        out_shape=[
            jax.ShapeDtypeStruct((HALF, D), jnp.bfloat16),
            jax.ShapeDtypeStruct((HALF, D), jnp.bfloat16),
        ],
        scratch_shapes=[
            pltpu.VMEM((CAP, D), jnp.float32),
            pltpu.VMEM((HALF, D), jnp.bfloat16),
            pltpu.SemaphoreType.DMA((E_LOCAL,)),
            pltpu.SemaphoreType.DMA((E_LOCAL,)),
            pltpu.SemaphoreType.DMA((E_LOCAL,)),
            pltpu.SemaphoreType.DMA((E_LOCAL,)),
            pltpu.SemaphoreType.DMA((E_LOCAL,)),
            pltpu.SemaphoreType.DMA((E_LOCAL,)),
        ],
        compiler_params=pltpu.CompilerParams(
            collective_id=1, vmem_limit_bytes=100 * 1024 * 1024
        ),
    )(rows_half, w1, w2)


def _combine_body(ts_ref, m_ref, r_ref, out_ref):
    cols = lax.broadcasted_iota(jnp.int32, (1, HALF), 1)
    ts = ts_ref[...]
    pm = (cols == ts).astype(jnp.bfloat16)
    pr = (cols == ts - HALF).astype(jnp.bfloat16)
    out_ref[...] = jnp.dot(
        pm, m_ref[...], preferred_element_type=jnp.float32
    ) + jnp.dot(pr, r_ref[...], preferred_element_type=jnp.float32)


def _combine(out_mine, returned, ts_col):
    return pl.pallas_call(
        _combine_body,
        grid=(T // TTILE,),
        in_specs=[
            pl.BlockSpec((TTILE, 1), lambda j: (j, 0)),
            pl.BlockSpec((HALF, D), lambda j: (0, 0)),
            pl.BlockSpec((HALF, D), lambda j: (0, 0)),
        ],
        out_specs=pl.BlockSpec((TTILE, D), lambda j: (j, 0)),
        out_shape=jax.ShapeDtypeStruct((T, D), jnp.float32),
        compiler_params=pltpu.CompilerParams(
            vmem_limit_bytes=100 * 1024 * 1024
        ),
    )(ts_col, out_mine, returned)


def kernel(x, assign, W1, W2):
    my_y = lax.axis_index("y")

    onehot = (
        assign[:, None] == jnp.arange(E_TOTAL, dtype=jnp.int32)[None, :]
    ).astype(jnp.int32)
    ranks = jnp.sum(onehot * jnp.cumsum(onehot, axis=0), axis=1) - 1
    section = jnp.where(assign // E_LOCAL == my_y, 0, HALF)
    token_slot = section + (assign % E_LOCAL) * CAP + ranks

    rows_all = _dispatch(
        x.astype(jnp.bfloat16), token_slot.reshape(1, T)
    ).reshape(E_LOCAL, 2 * CAP, D)

    my_x = lax.axis_index("x")
    rows_half = jnp.where(
        my_x == 0, rows_all[:, :CAP, :], rows_all[:, CAP:, :]
    )

    out_mine, returned = _moe(rows_half, W1, W2)

    return _combine(out_mine, returned, token_slot.reshape(T, 1))


# device time: 414496 ns/iter; 1.1142x vs baseline; 1.0049x over previous
import jax
import jax.numpy as jnp
from jax import lax
from jax.experimental import pallas as pl
from jax.experimental.pallas import tpu as pltpu

T = 4096
D = 2048
F = 4096
E_LOCAL = 4
E_TOTAL = 8
CAP = 544
HALF = E_LOCAL * CAP
FTILE = 512
STILE = CAP
TTILE = 1024


def _dispatch_body(
    ts_ref, x_ref, rows_ref, theirs_buf, send_sems, recv_sems
):
    peer = (lax.axis_index("x"), 1 - lax.axis_index("y"))
    barrier = pltpu.get_barrier_semaphore()
    pl.semaphore_signal(
        barrier, inc=1, device_id=peer, device_id_type=pl.DeviceIdType.MESH
    )
    pl.semaphore_wait(barrier, 1)

    nt = HALF // STILE

    def onehot_rows(base, t):
        rows = (base + t * STILE) + lax.broadcasted_iota(
            jnp.int32, (STILE, 1), 0
        )
        return (rows == ts_ref[...]).astype(jnp.bfloat16)

    rdmas = []
    for t in range(nt):
        theirs_buf[pl.ds(t * STILE, STILE), :] = jnp.dot(
            onehot_rows(HALF, t), x_ref[...],
            preferred_element_type=jnp.float32,
        ).astype(jnp.bfloat16)
        rdma = pltpu.make_async_remote_copy(
            src_ref=theirs_buf.at[pl.ds(t * STILE, STILE), :],
            dst_ref=rows_ref.at[pl.ds(t * 2 * CAP + CAP, CAP), :],
            send_sem=send_sems.at[t],
            recv_sem=recv_sems.at[t],
            device_id=peer,
            device_id_type=pl.DeviceIdType.MESH,
        )
        rdma.start()
        rdmas.append(rdma)

    def onehot_mine(t):
        rows = t * 2 * STILE + lax.broadcasted_iota(
            jnp.int32, (2 * STILE, 1), 0
        )
        return (rows == ts_ref[...]).astype(jnp.bfloat16)

    for t in range(nt // 2):
        pair = jnp.dot(
            onehot_mine(t), x_ref[...],
            preferred_element_type=jnp.float32,
        ).astype(jnp.bfloat16)
        rows_ref[pl.ds((2 * t) * 2 * CAP, CAP), :] = pair[:CAP, :]
        rows_ref[pl.ds((2 * t + 1) * 2 * CAP, CAP), :] = pair[CAP:, :]

    for rdma in rdmas:
        rdma.wait()


def _dispatch(x_bf, ts_row):
    return pl.pallas_call(
        _dispatch_body,
        out_shape=jax.ShapeDtypeStruct((2 * HALF, D), jnp.bfloat16),
        in_specs=[
            pl.BlockSpec(memory_space=pltpu.VMEM),
            pl.BlockSpec(memory_space=pltpu.VMEM),
        ],
        out_specs=pl.BlockSpec(memory_space=pltpu.VMEM),
        scratch_shapes=[
            pltpu.VMEM((HALF, D), jnp.bfloat16),
            pltpu.SemaphoreType.DMA((HALF // STILE,)),
            pltpu.SemaphoreType.DMA((HALF // STILE,)),
        ],
        compiler_params=pltpu.CompilerParams(
            collective_id=0, vmem_limit_bytes=100 * 1024 * 1024
        ),
    )(ts_row, x_bf)


def _flow(src_ref, src_row, dst_ref, dst_row, ssem, rsem, target, e):
    return pltpu.make_async_remote_copy(
        src_ref=src_ref.at[pl.ds(src_row, CAP), :],
        dst_ref=dst_ref.at[pl.ds(dst_row, CAP), :],
        send_sem=ssem.at[e],
        recv_sem=rsem.at[e],
        device_id=target,
        device_id_type=pl.DeviceIdType.MESH,
    )


def _moe_body(
    rows_ref,
    w1_ref,
    w2_ref,
    outm_ref,
    returned_ref,
    acc,
    retbuf,
    s1,
    r1,
    s2,
    r2,
    s3,
    r3,
):
    e = pl.program_id(0)
    ft = pl.program_id(1)
    my_x = lax.axis_index("x")
    my_y = lax.axis_index("y")
    xnbr = (1 - my_x, my_y)
    ypeer = (my_x, 1 - my_y)
    last = F // FTILE - 1

    @pl.when((e == 0) & (ft == 0))
    def _():
        barrier = pltpu.get_barrier_semaphore()
        for tgt in (xnbr, ypeer):
            pl.semaphore_signal(
                barrier,
                inc=1,
                device_id=tgt,
                device_id_type=pl.DeviceIdType.MESH,
            )
        pl.semaphore_wait(barrier, 2)

    w1 = w1_ref[0].astype(jnp.bfloat16)
    w2 = w2_ref[0].astype(jnp.bfloat16)

    h = jnp.dot(rows_ref[0], w1, preferred_element_type=jnp.float32)
    p = jnp.dot(
        jnp.maximum(h, 0.0).astype(jnp.bfloat16),
        w2,
        preferred_element_type=jnp.float32,
    )

    @pl.when(ft == 0)
    def _():
        acc[...] = p

    @pl.when(ft != 0)
    def _():
        acc[...] += p

    @pl.when(ft == last)
    def _():
        res = acc[...].astype(jnp.bfloat16)

        @pl.when(my_x == 0)
        def _():
            outm_ref[pl.ds(e * CAP, CAP), :] = res
            _flow(outm_ref, e * CAP, outm_ref, e * CAP, s1, r1, xnbr, e).start()

        @pl.when(my_x == 1)
        def _():
            retbuf[pl.ds(e * CAP, CAP), :] = res
            _flow(
                retbuf, e * CAP, returned_ref, e * CAP, s2, r2, ypeer, e
            ).start()

            @pl.when(e > 0)
            def _():
                prev = (e - 1) * CAP
                _flow(
                    retbuf, prev, returned_ref, prev, s2, r2, ypeer, e - 1
                ).wait_recv()
                _flow(
                    returned_ref, prev, returned_ref, prev, s3, r3, xnbr,
                    e - 1,
                ).start()

    @pl.when((e == E_LOCAL - 1) & (ft == last))
    def _():
        @pl.when(my_x == 0)
        def _():
            for t in range(E_LOCAL):
                row = t * CAP
                _flow(
                    outm_ref, row, outm_ref, row, s1, r1, xnbr, t
                ).wait_send()
                _flow(
                    returned_ref, row, returned_ref, row, s3, r3, xnbr, t
                ).wait_recv()

        @pl.when(my_x == 1)
        def _():
            row = (E_LOCAL - 1) * CAP
            _flow(
                retbuf, row, returned_ref, row, s2, r2, ypeer, E_LOCAL - 1
            ).wait_recv()
            _flow(
                returned_ref, row, returned_ref, row, s3, r3, xnbr,
                E_LOCAL - 1,
            ).start()
            for t in range(E_LOCAL):
                row = t * CAP
                _flow(
                    retbuf, row, returned_ref, row, s2, r2, ypeer, t
                ).wait_send()
                _flow(
                    returned_ref, row, returned_ref, row, s3, r3, xnbr, t
                ).wait_send()
                _flow(
                    outm_ref, row, outm_ref, row, s1, r1, xnbr, t
                ).wait_recv()


def _moe(rows_half, w1, w2):
    return pl.pallas_call(
        _moe_body,
        grid=(E_LOCAL, F // FTILE),
        in_specs=[
            pl.BlockSpec((1, CAP, D), lambda e, f: (e, 0, 0)),
            pl.BlockSpec((1, D, FTILE), lambda e, f: (e, 0, f)),
            pl.BlockSpec((1, FTILE, D), lambda e, f: (e, f, 0)),
        ],
        out_specs=[
            pl.BlockSpec(memory_space=pltpu.VMEM),
            pl.BlockSpec(memory_space=pltpu.VMEM),
        ],
        out_shape=[
            jax.ShapeDtypeStruct((HALF, D), jnp.bfloat16),
            jax.ShapeDtypeStruct((HALF, D), jnp.bfloat16),
        ],
        scratch_shapes=[
            pltpu.VMEM((CAP, D), jnp.float32),
            pltpu.VMEM((HALF, D), jnp.bfloat16),
            pltpu.SemaphoreType.DMA((E_LOCAL,)),
            pltpu.SemaphoreType.DMA((E_LOCAL,)),
            pltpu.SemaphoreType.DMA((E_LOCAL,)),
            pltpu.SemaphoreType.DMA((E_LOCAL,)),
            pltpu.SemaphoreType.DMA((E_LOCAL,)),
            pltpu.SemaphoreType.DMA((E_LOCAL,)),
        ],
        compiler_params=pltpu.CompilerParams(
            collective_id=1, vmem_limit_bytes=100 * 1024 * 1024
        ),
    )(rows_half, w1, w2)


def _combine_body(ts_ref, m_ref, r_ref, out_ref):
    cols = lax.broadcasted_iota(jnp.int32, (1, HALF), 1)
    ts = ts_ref[...]
    pm = (cols == ts).astype(jnp.bfloat16)
    pr = (cols == ts - HALF).astype(jnp.bfloat16)
    out_ref[...] = jnp.dot(
        pm, m_ref[...], preferred_element_type=jnp.float32
    ) + jnp.dot(pr, r_ref[...], preferred_element_type=jnp.float32)


def _combine(out_mine, returned, ts_col):
    return pl.pallas_call(
        _combine_body,
        grid=(T // TTILE,),
        in_specs=[
            pl.BlockSpec((TTILE, 1), lambda j: (j, 0)),
            pl.BlockSpec((HALF, D), lambda j: (0, 0)),
            pl.BlockSpec((HALF, D), lambda j: (0, 0)),
        ],
        out_specs=pl.BlockSpec((TTILE, D), lambda j: (j, 0)),
        out_shape=jax.ShapeDtypeStruct((T, D), jnp.float32),
        compiler_params=pltpu.CompilerParams(
            vmem_limit_bytes=100 * 1024 * 1024
        ),
    )(ts_col, out_mine, returned)


def kernel(x, assign, W1, W2):
    my_y = lax.axis_index("y")

    onehot = (
        assign[:, None] == jnp.arange(E_TOTAL, dtype=jnp.int32)[None, :]
    ).astype(jnp.int32)
    ranks = jnp.sum(onehot * jnp.cumsum(onehot, axis=0), axis=1) - 1
    section = jnp.where(assign // E_LOCAL == my_y, 0, HALF)
    token_slot = section + (assign % E_LOCAL) * CAP + ranks

    rows_all = _dispatch(
        x.astype(jnp.bfloat16), token_slot.reshape(1, T)
    ).reshape(E_LOCAL, 2 * CAP, D)

    my_x = lax.axis_index("x")
    rows_half = lax.dynamic_slice(
        rows_all, (0, my_x * CAP, 0), (E_LOCAL, CAP, D)
    )

    out_mine, returned = _moe(rows_half, W1, W2)

    return _combine(out_mine, returned, token_slot.reshape(T, 1))
